# baseline (speedup 1.0000x reference)
import jax
import jax.numpy as jnp
from jax import lax
from jax.experimental import pallas as pl
from jax.experimental.pallas import tpu as pltpu
from jax.experimental.pallas import tpu_sc as plsc

_NC, _NS, _L = 2, 16, 16
_NW = _NC * _NS
_CR = 16
_UNROLL = 8


def _sc_body(B, S, E, x_hbm, pe_hbm, out_hbm, pe_buf, x_buf,
             sem_in0, sem_in1, sem_out0, sem_out1):
    sem_in = (sem_in0, sem_in1)
    sem_out = (sem_out0, sem_out1)
    rows_w = S // _NW
    nch = rows_w // _CR
    ce = _CR * E
    w = lax.axis_index("s") * _NC + lax.axis_index("c")
    s0 = w * rows_w

    def issue_in(k, slot):
        base = s0 + k * _CR
        d = [pltpu.async_copy(pe_hbm.at[pl.ds(base * E, ce)],
                              pe_buf.at[slot], sem_in[slot])]
        for b in range(B):
            d.append(pltpu.async_copy(
                x_hbm.at[pl.ds((b * S + base) * E, ce)],
                x_buf.at[slot, b], sem_in[slot]))
        return d

    def issue_out(k, slot):
        base = s0 + k * _CR
        return [pltpu.async_copy(
            x_buf.at[slot, b],
            out_hbm.at[pl.ds((b * S + base) * E, ce)],
            sem_out[slot]) for b in range(B)]

    def compute(slot):
        def body(i, carry):
            for u in range(_UNROLL):
                off = (i * _UNROLL + u) * _L
                pv = pe_buf[slot, pl.ds(off, _L)]
                for b in range(B):
                    plsc.addupdate(x_buf.at[slot, b, pl.ds(off, _L)], pv)
            return carry
        lax.fori_loop(0, ce // (_L * _UNROLL), body, 0)

    pend_out = [None, None]
    pend_in = issue_in(0, 0)
    for k in range(nch):
        slot = k % 2
        nslot = (k + 1) % 2
        if k + 1 < nch:
            if pend_out[nslot] is not None:
                for d in pend_out[nslot]:
                    d.wait()
            next_in = issue_in(k + 1, nslot)
        else:
            next_in = None
        for d in pend_in:
            d.wait()
        compute(slot)
        pend_out[slot] = issue_out(k, slot)
        pend_in = next_in
    for ds in pend_out:
        if ds is not None:
            for d in ds:
                d.wait()


def kernel(x, pe):
    B, S, E = x.shape
    x1 = x.reshape(B * S * E)
    pe1 = pe.reshape(S * E)
    mesh = plsc.VectorSubcoreMesh(
        core_axis_name="c", subcore_axis_name="s",
        num_cores=_NC, num_subcores=_NS)
    ce = _CR * E
    body = lambda *refs: _sc_body(B, S, E, *refs)
    out1 = pl.kernel(
        body,
        out_type=jax.ShapeDtypeStruct((B * S * E,), x.dtype),
        mesh=mesh,
        scratch_types=[
            pltpu.VMEM((2, ce), jnp.float32),
            pltpu.VMEM((2, B, ce), jnp.float32),
            pltpu.SemaphoreType.DMA,
            pltpu.SemaphoreType.DMA,
            pltpu.SemaphoreType.DMA,
            pltpu.SemaphoreType.DMA,
        ],
    )(x1, pe1)
    return out1.reshape(B, S, E)


# speedup vs baseline: 5.3669x; 5.3669x over previous
import jax
import jax.numpy as jnp
from jax.experimental import pallas as pl

_BLK = 512


def _add_pe_kernel(x_ref, pe_ref, o_ref):
    o_ref[...] = x_ref[...] + pe_ref[...][None, :, :]


def kernel(x, pe):
    B, S, E = x.shape
    blk = min(_BLK, S)
    grid = (S // blk,)
    return pl.pallas_call(
        _add_pe_kernel,
        grid=grid,
        in_specs=[
            pl.BlockSpec((B, blk, E), lambda i: (0, i, 0)),
            pl.BlockSpec((blk, E), lambda i: (i, 0)),
        ],
        out_specs=pl.BlockSpec((B, blk, E), lambda i: (0, i, 0)),
        out_shape=jax.ShapeDtypeStruct((B, S, E), x.dtype),
    )(x, pe)
